# baseline (speedup 1.0000x reference)
import jax
import jax.numpy as jnp
from jax.experimental import pallas as pl
from jax.experimental.pallas import tpu as pltpu


def _node_project(embeddings, W1, b1, Npad, BN):
    N, D = embeddings.shape
    H = W1.shape[1]
    embp = jnp.pad(embeddings, ((0, Npad - N), (0, 0)))
    W1G = jnp.concatenate([W1[:D], W1[D:2 * D]], axis=1)
    W1G = jnp.pad(W1G, ((0, 0), (0, D - 2 * H)))
    b1p = jnp.pad(b1, (0, D - H)).reshape(1, D)

    def nodek(emb_ref, w_ref, b_ref, g_ref):
        g_ref[:] = (
            jnp.dot(emb_ref[:], w_ref[:], preferred_element_type=jnp.float32)
            + b_ref[:]
        )

    NBN = Npad // BN

    G = pl.pallas_call(
        nodek,
        grid=(2, NBN // 2),
        in_specs=[
            pl.BlockSpec((BN, D), lambda c, i: (c * (NBN // 2) + i, 0)),
            pl.BlockSpec((D, D), lambda c, i: (0, 0)),
            pl.BlockSpec((1, D), lambda c, i: (0, 0)),
        ],
        out_specs=pl.BlockSpec((BN, D), lambda c, i: (c * (NBN // 2) + i, 0)),
        out_shape=jax.ShapeDtypeStruct((Npad, D), jnp.float32),
        compiler_params=pltpu.CompilerParams(
            dimension_semantics=("parallel", "arbitrary"),
        ),
        name="node_project",
    )(embp, W1G, b1p)
    return G


def kernel(embeddings, edge_attr, edge_index, W1, b1, W2, b2, W3, b3, W4, b4):
    N, D = embeddings.shape
    E, F = edge_attr.shape
    H = W2.shape[0]

    for BM in (3200, 1280, 640, 256, 128):
        if E % (2 * BM) == 0:
            break
    NBI = E // (2 * BM)

    BN = 512
    Npad = -(-N // (2 * BN)) * (2 * BN)

    G3 = _node_project(embeddings, W1, b1, Npad, BN).reshape(Npad, 1, D)

    src = edge_index[0].astype(jnp.uint32)
    dst = edge_index[1].astype(jnp.uint32)
    idxp = jax.lax.bitcast_convert_type(
        src | (dst << 16), jnp.int32
    ).reshape(2, NBI, BM)
    idxp = jnp.pad(idxp, ((0, 0), (0, 1), (0, 0)))

    W1c = W1[2 * D:]
    eye = jnp.eye(H, dtype=jnp.float32)
    J = jnp.concatenate(
        [eye, eye, jnp.zeros((D - 2 * H, H), jnp.float32)], axis=0
    )
    b2r = b2.reshape(1, H)
    b3r = b3.reshape(1, H)
    b4r = b4.reshape(1, 1)

    U = 32
    SLOTW = BM

    def edgek(idx_hbm, attr_ref, g_ref, j_ref, w1c_ref, w2_ref, b2_ref,
              w3_ref, b3_ref, w4_ref, b4_ref, out_ref, x_scr, idx_smem,
              sems):
        gi = pl.program_id(1)
        o = pl.program_id(0)
        slot = jax.lax.rem(gi, 2)
        nslot = 1 - slot

        @pl.when(gi == 0)
        def _():
            pltpu.make_async_copy(
                idx_hbm.at[o, 0], idx_smem.at[pl.ds(0, BM)], sems.at[0]
            ).start()

        @pl.when(gi + 1 <= NBI)
        def _():
            pltpu.make_async_copy(
                idx_hbm.at[o, gi + 1],
                idx_smem.at[pl.ds(nslot * SLOTW, BM)],
                sems.at[nslot],
            ).start()

        pltpu.make_async_copy(
            idx_hbm.at[o, gi],
            idx_smem.at[pl.ds(slot * SLOTW, BM)],
            sems.at[slot],
        ).wait()
        off0 = slot * SLOTW

        lmask = jax.lax.broadcasted_iota(jnp.int32, (1, D), 1) < H

        def body(c, carry):
            base = c * U
            for q in range(U // 8):
                rows = []
                for u in range(8):
                    p = idx_smem[off0 + base + q * 8 + u]
                    i = p & 0xFFFF
                    j = jax.lax.shift_right_logical(p, 16)
                    a = g_ref[i]
                    b = g_ref[j]
                    rows.append(jnp.where(lmask, a, b))
                x_scr[slot, pl.ds(pl.multiple_of(base + q * 8, 8), 8), :] = (
                    jnp.concatenate(rows, axis=0)
                )
            return carry

        jax.lax.fori_loop(0, BM // U, body, 0)

        NCH = 4
        RC = BM // NCH
        for ch in range(NCH):
            rb = ch * RC
            xs = x_scr[nslot, pl.ds(rb, RC), :]
            h1 = jnp.maximum(
                jnp.dot(xs, j_ref[:], preferred_element_type=jnp.float32)
                + jnp.dot(attr_ref[pl.ds(rb, RC), :], w1c_ref[:],
                          preferred_element_type=jnp.float32),
                0.0,
            )
            h2 = jnp.maximum(
                jnp.dot(h1, w2_ref[:], preferred_element_type=jnp.float32)
                + b2_ref[:],
                0.0,
            )
            h3 = jnp.maximum(
                jnp.dot(h2, w3_ref[:], preferred_element_type=jnp.float32)
                + b3_ref[:],
                0.0,
            )
            out_ref[pl.ds(rb, RC), :] = (
                jnp.dot(h3, w4_ref[:], preferred_element_type=jnp.float32)
                + b4_ref[:]
            )

    def _prev(o, g):
        return o * NBI + jnp.clip(g - 1, 0, NBI - 1)

    out = pl.pallas_call(
        edgek,
        grid=(2, NBI + 1),
        in_specs=[
            pl.BlockSpec(memory_space=pl.ANY),
            pl.BlockSpec((BM, F), lambda o, g: (_prev(o, g), 0)),
            pl.BlockSpec((Npad, 1, D), lambda o, g: (0, 0, 0)),
            pl.BlockSpec((D, H), lambda o, g: (0, 0)),
            pl.BlockSpec((F, H), lambda o, g: (0, 0)),
            pl.BlockSpec((H, H), lambda o, g: (0, 0)),
            pl.BlockSpec((1, H), lambda o, g: (0, 0)),
            pl.BlockSpec((H, H), lambda o, g: (0, 0)),
            pl.BlockSpec((1, H), lambda o, g: (0, 0)),
            pl.BlockSpec((H, 1), lambda o, g: (0, 0)),
            pl.BlockSpec((1, 1), lambda o, g: (0, 0)),
        ],
        out_specs=pl.BlockSpec((BM, 1), lambda o, g: (_prev(o, g), 0)),
        out_shape=jax.ShapeDtypeStruct((E, 1), jnp.float32),
        scratch_shapes=[
            pltpu.VMEM((2, BM, D), jnp.float32),
            pltpu.SMEM((2 * SLOTW,), jnp.int32),
            pltpu.SemaphoreType.DMA((2,)),
        ],
        compiler_params=pltpu.CompilerParams(
            dimension_semantics=("parallel", "arbitrary"),
            vmem_limit_bytes=48 * 1024 * 1024,
        ),
        name="edge_mlp",
    )(idxp, edge_attr, G3, J, W1c, W2, b2r, W3, b3r, W4, b4r)

    ER = E // 128

    def sigk(l_ref, o_ref):
        o_ref[:] = jax.nn.sigmoid(l_ref[:])

    outs = pl.pallas_call(
        sigk,
        grid=(1,),
        in_specs=[pl.BlockSpec((ER, 128), lambda i: (0, 0))],
        out_specs=pl.BlockSpec((ER, 128), lambda i: (0, 0)),
        out_shape=jax.ShapeDtypeStruct((ER, 128), jnp.float32),
        compiler_params=pltpu.CompilerParams(
            vmem_limit_bytes=40 * 1024 * 1024,
        ),
        name="edge_sigmoid",
    )(out.reshape(ER, 128))
    return outs.reshape(E, 1)


# speedup vs baseline: 1.0640x; 1.0640x over previous
import jax
import jax.numpy as jnp
from jax.experimental import pallas as pl
from jax.experimental.pallas import tpu as pltpu


def _node_project(embeddings, W1, b1, Npad, BN):
    N, D = embeddings.shape
    H = W1.shape[1]
    embp = jnp.pad(embeddings, ((0, Npad - N), (0, 0)))
    W1G = jnp.concatenate([W1[:D], W1[D:2 * D]], axis=1)
    W1G = jnp.pad(W1G, ((0, 0), (0, D - 2 * H)))
    b1p = jnp.pad(b1, (0, D - H)).reshape(1, D)

    def nodek(emb_ref, w_ref, b_ref, g_ref):
        g_ref[:] = (
            jnp.dot(emb_ref[:], w_ref[:], preferred_element_type=jnp.float32)
            + b_ref[:]
        )

    NBN = Npad // BN

    G = pl.pallas_call(
        nodek,
        grid=(2, NBN // 2),
        in_specs=[
            pl.BlockSpec((BN, D), lambda c, i: (c * (NBN // 2) + i, 0)),
            pl.BlockSpec((D, D), lambda c, i: (0, 0)),
            pl.BlockSpec((1, D), lambda c, i: (0, 0)),
        ],
        out_specs=pl.BlockSpec((BN, D), lambda c, i: (c * (NBN // 2) + i, 0)),
        out_shape=jax.ShapeDtypeStruct((Npad, D), jnp.float32),
        compiler_params=pltpu.CompilerParams(
            dimension_semantics=("parallel", "arbitrary"),
        ),
        name="node_project",
    )(embp, W1G, b1p)
    return G


def kernel(embeddings, edge_attr, edge_index, W1, b1, W2, b2, W3, b3, W4, b4):
    N, D = embeddings.shape
    E, F = edge_attr.shape
    H = W2.shape[0]

    for BM in (3200, 1280, 640, 256, 128):
        if E % (2 * BM) == 0:
            break
    NBI = E // (2 * BM)

    BN = 512
    Npad = -(-N // (2 * BN)) * (2 * BN)

    G3 = _node_project(embeddings, W1, b1, Npad, BN).reshape(Npad, 1, D)

    src = edge_index[0].astype(jnp.uint32)
    dst = edge_index[1].astype(jnp.uint32)
    idxp = jax.lax.bitcast_convert_type(
        src | (dst << 16), jnp.int32
    ).reshape(2, NBI, BM)
    idxp = jnp.pad(idxp, ((0, 0), (0, 1), (0, 0)))

    W1c = W1[2 * D:]
    eye = jnp.eye(H, dtype=jnp.float32)
    J = jnp.concatenate(
        [eye, eye, jnp.zeros((D - 2 * H, H), jnp.float32)], axis=0
    )
    b2r = b2.reshape(1, H)
    b3r = b3.reshape(1, H)
    b4r = b4.reshape(1, 1)

    U = 32
    SLOTW = BM

    def edgek(idx_hbm, attr_ref, g_ref, j_ref, w1c_ref, w2_ref, b2_ref,
              w3_ref, b3_ref, w4_ref, b4_ref, out_ref, x_scr, idx_smem,
              sems):
        gi = pl.program_id(1)
        o = pl.program_id(0)
        slot = jax.lax.rem(gi, 2)
        nslot = 1 - slot

        @pl.when(gi == 0)
        def _():
            pltpu.make_async_copy(
                idx_hbm.at[o, 0], idx_smem.at[pl.ds(0, BM)], sems.at[0]
            ).start()

        @pl.when(gi + 1 <= NBI)
        def _():
            pltpu.make_async_copy(
                idx_hbm.at[o, gi + 1],
                idx_smem.at[pl.ds(nslot * SLOTW, BM)],
                sems.at[nslot],
            ).start()

        pltpu.make_async_copy(
            idx_hbm.at[o, gi],
            idx_smem.at[pl.ds(slot * SLOTW, BM)],
            sems.at[slot],
        ).wait()
        off0 = slot * SLOTW

        lmask = jax.lax.broadcasted_iota(jnp.int32, (1, D), 1) < H

        def body(c, carry):
            base = c * U
            for q in range(U // 8):
                rows = []
                for u in range(8):
                    p = idx_smem[off0 + base + q * 8 + u]
                    i = p & 0xFFFF
                    j = jax.lax.shift_right_logical(p, 16)
                    a = g_ref[i]
                    b = g_ref[j]
                    rows.append(jnp.where(lmask, a, b))
                x_scr[slot, pl.ds(pl.multiple_of(base + q * 8, 8), 8), :] = (
                    jnp.concatenate(rows, axis=0)
                )
            return carry

        jax.lax.fori_loop(0, BM // U, body, 0)

        xs = x_scr[nslot]
        h1 = jnp.maximum(
            jnp.dot(xs, j_ref[:], preferred_element_type=jnp.float32)
            + jnp.dot(attr_ref[:], w1c_ref[:],
                      preferred_element_type=jnp.float32),
            0.0,
        )
        h2 = jnp.maximum(
            jnp.dot(h1, w2_ref[:], preferred_element_type=jnp.float32)
            + b2_ref[:],
            0.0,
        )
        h3 = jnp.maximum(
            jnp.dot(h2, w3_ref[:], preferred_element_type=jnp.float32)
            + b3_ref[:],
            0.0,
        )
        out_ref[:] = (
            jnp.dot(h3, w4_ref[:], preferred_element_type=jnp.float32)
            + b4_ref[:]
        )

    def _prev(o, g):
        return o * NBI + jnp.clip(g - 1, 0, NBI - 1)

    out = pl.pallas_call(
        edgek,
        grid=(2, NBI + 1),
        in_specs=[
            pl.BlockSpec(memory_space=pl.ANY),
            pl.BlockSpec((BM, F), lambda o, g: (_prev(o, g), 0)),
            pl.BlockSpec((Npad, 1, D), lambda o, g: (0, 0, 0)),
            pl.BlockSpec((D, H), lambda o, g: (0, 0)),
            pl.BlockSpec((F, H), lambda o, g: (0, 0)),
            pl.BlockSpec((H, H), lambda o, g: (0, 0)),
            pl.BlockSpec((1, H), lambda o, g: (0, 0)),
            pl.BlockSpec((H, H), lambda o, g: (0, 0)),
            pl.BlockSpec((1, H), lambda o, g: (0, 0)),
            pl.BlockSpec((H, 1), lambda o, g: (0, 0)),
            pl.BlockSpec((1, 1), lambda o, g: (0, 0)),
        ],
        out_specs=pl.BlockSpec((BM, 1), lambda o, g: (_prev(o, g), 0)),
        out_shape=jax.ShapeDtypeStruct((E, 1), jnp.float32),
        scratch_shapes=[
            pltpu.VMEM((2, BM, D), jnp.float32),
            pltpu.SMEM((2 * SLOTW,), jnp.int32),
            pltpu.SemaphoreType.DMA((2,)),
        ],
        compiler_params=pltpu.CompilerParams(
            dimension_semantics=("parallel", "arbitrary"),
            vmem_limit_bytes=48 * 1024 * 1024,
        ),
        name="edge_mlp",
    )(idxp, edge_attr, G3, J, W1c, W2, b2r, W3, b3r, W4, b4r)

    ER = E // 128

    def sigk(l_ref, o_ref):
        o_ref[:] = jax.nn.sigmoid(l_ref[:])

    outs = pl.pallas_call(
        sigk,
        grid=(1,),
        in_specs=[pl.BlockSpec((ER, 128), lambda i: (0, 0))],
        out_specs=pl.BlockSpec((ER, 128), lambda i: (0, 0)),
        out_shape=jax.ShapeDtypeStruct((ER, 128), jnp.float32),
        compiler_params=pltpu.CompilerParams(
            vmem_limit_bytes=40 * 1024 * 1024,
        ),
        name="edge_sigmoid",
    )(out.reshape(ER, 128))
    return outs.reshape(E, 1)


# speedup vs baseline: 1.1362x; 1.0679x over previous
import jax
import jax.numpy as jnp
from jax.experimental import pallas as pl
from jax.experimental.pallas import tpu as pltpu


def _node_project(embeddings, W1, b1, Npad, BN):
    N, D = embeddings.shape
    H = W1.shape[1]
    embp = jnp.pad(embeddings, ((0, Npad - N), (0, 0)))
    W1G = jnp.concatenate([W1[:D], W1[D:2 * D]], axis=1)
    W1G = jnp.pad(W1G, ((0, 0), (0, D - 2 * H)))
    b1p = jnp.pad(b1, (0, D - H)).reshape(1, D)

    def nodek(emb_ref, w_ref, b_ref, g_ref):
        g_ref[:] = (
            jnp.dot(emb_ref[:], w_ref[:], preferred_element_type=jnp.float32)
            + b_ref[:]
        )

    NBN = Npad // BN

    G = pl.pallas_call(
        nodek,
        grid=(2, NBN // 2),
        in_specs=[
            pl.BlockSpec((BN, D), lambda c, i: (c * (NBN // 2) + i, 0)),
            pl.BlockSpec((D, D), lambda c, i: (0, 0)),
            pl.BlockSpec((1, D), lambda c, i: (0, 0)),
        ],
        out_specs=pl.BlockSpec((BN, D), lambda c, i: (c * (NBN // 2) + i, 0)),
        out_shape=jax.ShapeDtypeStruct((Npad, D), jnp.float32),
        compiler_params=pltpu.CompilerParams(
            dimension_semantics=("parallel", "arbitrary"),
        ),
        name="node_project",
    )(embp, W1G, b1p)
    return G


def kernel(embeddings, edge_attr, edge_index, W1, b1, W2, b2, W3, b3, W4, b4):
    N, D = embeddings.shape
    E, F = edge_attr.shape
    H = W2.shape[0]

    for BM in (6400, 3200, 1280, 640, 256, 128):
        if E % (2 * BM) == 0:
            break
    NBI = E // (2 * BM)

    BN = 512
    Npad = -(-N // (2 * BN)) * (2 * BN)

    G3 = _node_project(embeddings, W1, b1, Npad, BN).reshape(Npad, 1, D)

    src = edge_index[0].astype(jnp.uint32)
    dst = edge_index[1].astype(jnp.uint32)
    idxp = jax.lax.bitcast_convert_type(
        src | (dst << 16), jnp.int32
    ).reshape(2, NBI, BM)
    idxp = jnp.pad(idxp, ((0, 0), (0, 1), (0, 0)))

    W1c = W1[2 * D:]
    eye = jnp.eye(H, dtype=jnp.float32)
    J = jnp.concatenate(
        [eye, eye, jnp.zeros((D - 2 * H, H), jnp.float32)], axis=0
    )
    b2r = b2.reshape(1, H)
    b3r = b3.reshape(1, H)
    b4r = b4.reshape(1, 1)

    U = 64
    SLOTW = BM

    def edgek(idx_hbm, attr_ref, g_ref, j_ref, w1c_ref, w2_ref, b2_ref,
              w3_ref, b3_ref, w4_ref, b4_ref, out_ref, x_scr, idx_smem,
              sems):
        gi = pl.program_id(1)
        o = pl.program_id(0)
        slot = jax.lax.rem(gi, 2)
        nslot = 1 - slot

        @pl.when(gi == 0)
        def _():
            pltpu.make_async_copy(
                idx_hbm.at[o, 0], idx_smem.at[pl.ds(0, BM)], sems.at[0]
            ).start()

        @pl.when(gi + 1 <= NBI)
        def _():
            pltpu.make_async_copy(
                idx_hbm.at[o, gi + 1],
                idx_smem.at[pl.ds(nslot * SLOTW, BM)],
                sems.at[nslot],
            ).start()

        pltpu.make_async_copy(
            idx_hbm.at[o, gi],
            idx_smem.at[pl.ds(slot * SLOTW, BM)],
            sems.at[slot],
        ).wait()
        off0 = slot * SLOTW

        lmask = jax.lax.broadcasted_iota(jnp.int32, (1, D), 1) < H

        def body(c, carry):
            base = c * U
            for q in range(U // 8):
                rows = []
                for u in range(8):
                    p = idx_smem[off0 + base + q * 8 + u]
                    i = p & 0xFFFF
                    j = jax.lax.shift_right_logical(p, 16)
                    a = g_ref[i]
                    b = g_ref[j]
                    rows.append(jnp.where(lmask, a, b))
                x_scr[slot, pl.ds(pl.multiple_of(base + q * 8, 8), 8), :] = (
                    jnp.concatenate(rows, axis=0)
                )
            return carry

        jax.lax.fori_loop(0, BM // U, body, 0)

        xs = x_scr[nslot]
        h1 = jnp.maximum(
            jnp.dot(xs, j_ref[:], preferred_element_type=jnp.float32)
            + jnp.dot(attr_ref[:], w1c_ref[:],
                      preferred_element_type=jnp.float32),
            0.0,
        )
        h2 = jnp.maximum(
            jnp.dot(h1, w2_ref[:], preferred_element_type=jnp.float32)
            + b2_ref[:],
            0.0,
        )
        h3 = jnp.maximum(
            jnp.dot(h2, w3_ref[:], preferred_element_type=jnp.float32)
            + b3_ref[:],
            0.0,
        )
        out_ref[:] = (
            jnp.dot(h3, w4_ref[:], preferred_element_type=jnp.float32)
            + b4_ref[:]
        )

    def _prev(o, g):
        return o * NBI + jnp.clip(g - 1, 0, NBI - 1)

    out = pl.pallas_call(
        edgek,
        grid=(2, NBI + 1),
        in_specs=[
            pl.BlockSpec(memory_space=pl.ANY),
            pl.BlockSpec((BM, F), lambda o, g: (_prev(o, g), 0)),
            pl.BlockSpec((Npad, 1, D), lambda o, g: (0, 0, 0)),
            pl.BlockSpec((D, H), lambda o, g: (0, 0)),
            pl.BlockSpec((F, H), lambda o, g: (0, 0)),
            pl.BlockSpec((H, H), lambda o, g: (0, 0)),
            pl.BlockSpec((1, H), lambda o, g: (0, 0)),
            pl.BlockSpec((H, H), lambda o, g: (0, 0)),
            pl.BlockSpec((1, H), lambda o, g: (0, 0)),
            pl.BlockSpec((H, 1), lambda o, g: (0, 0)),
            pl.BlockSpec((1, 1), lambda o, g: (0, 0)),
        ],
        out_specs=pl.BlockSpec((BM, 1), lambda o, g: (_prev(o, g), 0)),
        out_shape=jax.ShapeDtypeStruct((E, 1), jnp.float32),
        scratch_shapes=[
            pltpu.VMEM((2, BM, D), jnp.float32),
            pltpu.SMEM((2 * SLOTW,), jnp.int32),
            pltpu.SemaphoreType.DMA((2,)),
        ],
        compiler_params=pltpu.CompilerParams(
            dimension_semantics=("parallel", "arbitrary"),
            vmem_limit_bytes=56 * 1024 * 1024,
        ),
        name="edge_mlp",
    )(idxp, edge_attr, G3, J, W1c, W2, b2r, W3, b3r, W4, b4r)

    ER = E // 128

    def sigk(l_ref, o_ref):
        o_ref[:] = jax.nn.sigmoid(l_ref[:])

    outs = pl.pallas_call(
        sigk,
        grid=(1,),
        in_specs=[pl.BlockSpec((ER, 128), lambda i: (0, 0))],
        out_specs=pl.BlockSpec((ER, 128), lambda i: (0, 0)),
        out_shape=jax.ShapeDtypeStruct((ER, 128), jnp.float32),
        compiler_params=pltpu.CompilerParams(
            vmem_limit_bytes=40 * 1024 * 1024,
        ),
        name="edge_sigmoid",
    )(out.reshape(ER, 128))
    return outs.reshape(E, 1)


# speedup vs baseline: 1.1792x; 1.0378x over previous
import jax
import jax.numpy as jnp
from jax.experimental import pallas as pl
from jax.experimental.pallas import tpu as pltpu


def _node_project(embeddings, W1, b1, Npad, BN):
    N, D = embeddings.shape
    H = W1.shape[1]
    embp = jnp.pad(embeddings, ((0, Npad - N), (0, 0)))
    W1G = jnp.concatenate([W1[:D], W1[D:2 * D]], axis=1)
    W1G = jnp.pad(W1G, ((0, 0), (0, D - 2 * H)))
    b1p = jnp.pad(b1, (0, D - H)).reshape(1, D)

    def nodek(emb_ref, w_ref, b_ref, g_ref):
        g_ref[:] = (
            jnp.dot(emb_ref[:], w_ref[:], preferred_element_type=jnp.float32)
            + b_ref[:]
        )

    NBN = Npad // BN

    G = pl.pallas_call(
        nodek,
        grid=(2, NBN // 2),
        in_specs=[
            pl.BlockSpec((BN, D), lambda c, i: (c * (NBN // 2) + i, 0)),
            pl.BlockSpec((D, D), lambda c, i: (0, 0)),
            pl.BlockSpec((1, D), lambda c, i: (0, 0)),
        ],
        out_specs=pl.BlockSpec((BN, D), lambda c, i: (c * (NBN // 2) + i, 0)),
        out_shape=jax.ShapeDtypeStruct((Npad, D), jnp.float32),
        compiler_params=pltpu.CompilerParams(
            dimension_semantics=("parallel", "arbitrary"),
        ),
        name="node_project",
    )(embp, W1G, b1p)
    return G


def kernel(embeddings, edge_attr, edge_index, W1, b1, W2, b2, W3, b3, W4, b4):
    N, D = embeddings.shape
    E, F = edge_attr.shape
    H = W2.shape[0]

    for BM in (6400, 3200, 1280, 640, 256, 128):
        if E % (2 * BM) == 0:
            break
    NBI = E // (2 * BM)

    BN = 512
    Npad = -(-N // (2 * BN)) * (2 * BN)

    G3 = _node_project(embeddings, W1, b1, Npad, BN).reshape(Npad, 1, D)

    src = edge_index[0].astype(jnp.uint32)
    dst = edge_index[1].astype(jnp.uint32)
    idxp = jax.lax.bitcast_convert_type(
        src | (dst << 16), jnp.int32
    ).reshape(2, NBI, BM)
    idxp = jnp.pad(idxp, ((0, 0), (0, 1), (0, 0)))

    W1c = W1[2 * D:]
    eye = jnp.eye(H, dtype=jnp.float32)
    J = jnp.concatenate(
        [eye, eye, jnp.zeros((D - 2 * H, H), jnp.float32)], axis=0
    )
    b2r = b2.reshape(1, H)
    b3r = b3.reshape(1, H)
    b4r = b4.reshape(1, 1)

    U = 128
    SLOTW = BM

    def edgek(idx_hbm, attr_ref, g_ref, j_ref, w1c_ref, w2_ref, b2_ref,
              w3_ref, b3_ref, w4_ref, b4_ref, out_ref, x_scr, idx_smem,
              sems):
        gi = pl.program_id(1)
        o = pl.program_id(0)
        slot = jax.lax.rem(gi, 2)
        nslot = 1 - slot

        @pl.when(gi == 0)
        def _():
            pltpu.make_async_copy(
                idx_hbm.at[o, 0], idx_smem.at[pl.ds(0, BM)], sems.at[0]
            ).start()

        @pl.when(gi + 1 <= NBI)
        def _():
            pltpu.make_async_copy(
                idx_hbm.at[o, gi + 1],
                idx_smem.at[pl.ds(nslot * SLOTW, BM)],
                sems.at[nslot],
            ).start()

        pltpu.make_async_copy(
            idx_hbm.at[o, gi],
            idx_smem.at[pl.ds(slot * SLOTW, BM)],
            sems.at[slot],
        ).wait()
        off0 = slot * SLOTW

        lmask = jax.lax.broadcasted_iota(jnp.int32, (1, D), 1) < H

        def body(c, carry):
            base = c * U
            for q in range(U // 8):
                rows = []
                for u in range(8):
                    p = idx_smem[off0 + base + q * 8 + u]
                    i = p & 0xFFFF
                    j = jax.lax.shift_right_logical(p, 16)
                    a = g_ref[i]
                    b = g_ref[j]
                    rows.append(jnp.where(lmask, a, b))
                x_scr[slot, pl.ds(pl.multiple_of(base + q * 8, 8), 8), :] = (
                    jnp.concatenate(rows, axis=0)
                )
            return carry

        jax.lax.fori_loop(0, BM // U, body, 0)

        xs = x_scr[nslot]
        h1 = jnp.maximum(
            jnp.dot(xs, j_ref[:], preferred_element_type=jnp.float32)
            + jnp.dot(attr_ref[:], w1c_ref[:],
                      preferred_element_type=jnp.float32),
            0.0,
        )
        h2 = jnp.maximum(
            jnp.dot(h1, w2_ref[:], preferred_element_type=jnp.float32)
            + b2_ref[:],
            0.0,
        )
        h3 = jnp.maximum(
            jnp.dot(h2, w3_ref[:], preferred_element_type=jnp.float32)
            + b3_ref[:],
            0.0,
        )
        out_ref[:] = (
            jnp.dot(h3, w4_ref[:], preferred_element_type=jnp.float32)
            + b4_ref[:]
        )

    def _prev(o, g):
        return o * NBI + jnp.clip(g - 1, 0, NBI - 1)

    out = pl.pallas_call(
        edgek,
        grid=(2, NBI + 1),
        in_specs=[
            pl.BlockSpec(memory_space=pl.ANY),
            pl.BlockSpec((BM, F), lambda o, g: (_prev(o, g), 0)),
            pl.BlockSpec((Npad, 1, D), lambda o, g: (0, 0, 0)),
            pl.BlockSpec((D, H), lambda o, g: (0, 0)),
            pl.BlockSpec((F, H), lambda o, g: (0, 0)),
            pl.BlockSpec((H, H), lambda o, g: (0, 0)),
            pl.BlockSpec((1, H), lambda o, g: (0, 0)),
            pl.BlockSpec((H, H), lambda o, g: (0, 0)),
            pl.BlockSpec((1, H), lambda o, g: (0, 0)),
            pl.BlockSpec((H, 1), lambda o, g: (0, 0)),
            pl.BlockSpec((1, 1), lambda o, g: (0, 0)),
        ],
        out_specs=pl.BlockSpec((BM, 1), lambda o, g: (_prev(o, g), 0)),
        out_shape=jax.ShapeDtypeStruct((E, 1), jnp.float32),
        scratch_shapes=[
            pltpu.VMEM((2, BM, D), jnp.float32),
            pltpu.SMEM((2 * SLOTW,), jnp.int32),
            pltpu.SemaphoreType.DMA((2,)),
        ],
        compiler_params=pltpu.CompilerParams(
            dimension_semantics=("parallel", "arbitrary"),
            vmem_limit_bytes=56 * 1024 * 1024,
        ),
        name="edge_mlp",
    )(idxp, edge_attr, G3, J, W1c, W2, b2r, W3, b3r, W4, b4r)

    ER = E // 128

    def sigk(l_ref, o_ref):
        o_ref[:] = jax.nn.sigmoid(l_ref[:])

    outs = pl.pallas_call(
        sigk,
        grid=(1,),
        in_specs=[pl.BlockSpec((ER, 128), lambda i: (0, 0))],
        out_specs=pl.BlockSpec((ER, 128), lambda i: (0, 0)),
        out_shape=jax.ShapeDtypeStruct((ER, 128), jnp.float32),
        compiler_params=pltpu.CompilerParams(
            vmem_limit_bytes=40 * 1024 * 1024,
        ),
        name="edge_sigmoid",
    )(out.reshape(ER, 128))
    return outs.reshape(E, 1)


# speedup vs baseline: 1.1949x; 1.0133x over previous
import jax
import jax.numpy as jnp
from jax.experimental import pallas as pl
from jax.experimental.pallas import tpu as pltpu


def _node_project(embeddings, W1, b1, Npad, BN):
    N, D = embeddings.shape
    H = W1.shape[1]
    embp = jnp.pad(embeddings, ((0, Npad - N), (0, 0)))
    W1G = jnp.concatenate([W1[:D], W1[D:2 * D]], axis=1)
    W1G = jnp.pad(W1G, ((0, 0), (0, D - 2 * H)))
    b1p = jnp.pad(b1, (0, D - H)).reshape(1, D)

    def nodek(emb_ref, w_ref, b_ref, g_ref):
        g_ref[:] = (
            jnp.dot(emb_ref[:], w_ref[:], preferred_element_type=jnp.float32)
            + b_ref[:]
        )

    NBN = Npad // BN

    G = pl.pallas_call(
        nodek,
        grid=(2, NBN // 2),
        in_specs=[
            pl.BlockSpec((BN, D), lambda c, i: (c * (NBN // 2) + i, 0)),
            pl.BlockSpec((D, D), lambda c, i: (0, 0)),
            pl.BlockSpec((1, D), lambda c, i: (0, 0)),
        ],
        out_specs=pl.BlockSpec((BN, D), lambda c, i: (c * (NBN // 2) + i, 0)),
        out_shape=jax.ShapeDtypeStruct((Npad, D), jnp.float32),
        compiler_params=pltpu.CompilerParams(
            dimension_semantics=("parallel", "arbitrary"),
        ),
        name="node_project",
    )(embp, W1G, b1p)
    return G


def kernel(embeddings, edge_attr, edge_index, W1, b1, W2, b2, W3, b3, W4, b4):
    N, D = embeddings.shape
    E, F = edge_attr.shape
    H = W2.shape[0]

    for BM in (6400, 3200, 1280, 640, 256, 128):
        if E % (2 * BM) == 0:
            break
    NBI = E // (2 * BM)

    BN = 512
    Npad = -(-N // (2 * BN)) * (2 * BN)

    G3 = _node_project(embeddings, W1, b1, Npad, BN).reshape(Npad, 1, D)

    src = edge_index[0].astype(jnp.uint32)
    dst = edge_index[1].astype(jnp.uint32)
    idxp = jax.lax.bitcast_convert_type(
        src | (dst << 16), jnp.int32
    ).reshape(2, NBI, BM)
    idxp = jnp.pad(idxp, ((0, 0), (0, 1), (0, 0)))

    W1c = W1[2 * D:]
    eye = jnp.eye(H, dtype=jnp.float32)
    J = jnp.concatenate(
        [eye, eye, jnp.zeros((D - 2 * H, H), jnp.float32)], axis=0
    )
    b2r = b2.reshape(1, H)
    b3r = b3.reshape(1, H)
    b4r = b4.reshape(1, 1)

    U = 256 if BM % 256 == 0 else 64
    SLOTW = BM

    def edgek(idx_hbm, attr_ref, g_ref, j_ref, w1c_ref, w2_ref, b2_ref,
              w3_ref, b3_ref, w4_ref, b4_ref, out_ref, x_scr, idx_smem,
              sems):
        gi = pl.program_id(1)
        o = pl.program_id(0)
        slot = jax.lax.rem(gi, 2)
        nslot = 1 - slot

        @pl.when(gi == 0)
        def _():
            pltpu.make_async_copy(
                idx_hbm.at[o, 0], idx_smem.at[pl.ds(0, BM)], sems.at[0]
            ).start()

        @pl.when(gi + 1 <= NBI)
        def _():
            pltpu.make_async_copy(
                idx_hbm.at[o, gi + 1],
                idx_smem.at[pl.ds(nslot * SLOTW, BM)],
                sems.at[nslot],
            ).start()

        pltpu.make_async_copy(
            idx_hbm.at[o, gi],
            idx_smem.at[pl.ds(slot * SLOTW, BM)],
            sems.at[slot],
        ).wait()
        off0 = slot * SLOTW

        lmask = jax.lax.broadcasted_iota(jnp.int32, (1, D), 1) < H

        def body(c, carry):
            base = c * U
            for q in range(U // 8):
                rows = []
                for u in range(8):
                    p = idx_smem[off0 + base + q * 8 + u]
                    i = p & 0xFFFF
                    j = jax.lax.shift_right_logical(p, 16)
                    a = g_ref[i]
                    b = g_ref[j]
                    rows.append(jnp.where(lmask, a, b))
                x_scr[slot, pl.ds(pl.multiple_of(base + q * 8, 8), 8), :] = (
                    jnp.concatenate(rows, axis=0)
                )
            return carry

        jax.lax.fori_loop(0, BM // U, body, 0)

        xs = x_scr[nslot]
        h1 = jnp.maximum(
            jnp.dot(xs, j_ref[:], preferred_element_type=jnp.float32)
            + jnp.dot(attr_ref[:], w1c_ref[:],
                      preferred_element_type=jnp.float32),
            0.0,
        )
        h2 = jnp.maximum(
            jnp.dot(h1, w2_ref[:], preferred_element_type=jnp.float32)
            + b2_ref[:],
            0.0,
        )
        h3 = jnp.maximum(
            jnp.dot(h2, w3_ref[:], preferred_element_type=jnp.float32)
            + b3_ref[:],
            0.0,
        )
        out_ref[:] = (
            jnp.dot(h3, w4_ref[:], preferred_element_type=jnp.float32)
            + b4_ref[:]
        )

    def _prev(o, g):
        return o * NBI + jnp.clip(g - 1, 0, NBI - 1)

    out = pl.pallas_call(
        edgek,
        grid=(2, NBI + 1),
        in_specs=[
            pl.BlockSpec(memory_space=pl.ANY),
            pl.BlockSpec((BM, F), lambda o, g: (_prev(o, g), 0)),
            pl.BlockSpec((Npad, 1, D), lambda o, g: (0, 0, 0)),
            pl.BlockSpec((D, H), lambda o, g: (0, 0)),
            pl.BlockSpec((F, H), lambda o, g: (0, 0)),
            pl.BlockSpec((H, H), lambda o, g: (0, 0)),
            pl.BlockSpec((1, H), lambda o, g: (0, 0)),
            pl.BlockSpec((H, H), lambda o, g: (0, 0)),
            pl.BlockSpec((1, H), lambda o, g: (0, 0)),
            pl.BlockSpec((H, 1), lambda o, g: (0, 0)),
            pl.BlockSpec((1, 1), lambda o, g: (0, 0)),
        ],
        out_specs=pl.BlockSpec((BM, 1), lambda o, g: (_prev(o, g), 0)),
        out_shape=jax.ShapeDtypeStruct((E, 1), jnp.float32),
        scratch_shapes=[
            pltpu.VMEM((2, BM, D), jnp.float32),
            pltpu.SMEM((2 * SLOTW,), jnp.int32),
            pltpu.SemaphoreType.DMA((2,)),
        ],
        compiler_params=pltpu.CompilerParams(
            dimension_semantics=("parallel", "arbitrary"),
            vmem_limit_bytes=56 * 1024 * 1024,
        ),
        name="edge_mlp",
    )(idxp, edge_attr, G3, J, W1c, W2, b2r, W3, b3r, W4, b4r)

    ER = E // 128

    def sigk(l_ref, o_ref):
        o_ref[:] = jax.nn.sigmoid(l_ref[:])

    outs = pl.pallas_call(
        sigk,
        grid=(1,),
        in_specs=[pl.BlockSpec((ER, 128), lambda i: (0, 0))],
        out_specs=pl.BlockSpec((ER, 128), lambda i: (0, 0)),
        out_shape=jax.ShapeDtypeStruct((ER, 128), jnp.float32),
        compiler_params=pltpu.CompilerParams(
            vmem_limit_bytes=40 * 1024 * 1024,
        ),
        name="edge_sigmoid",
    )(out.reshape(ER, 128))
    return outs.reshape(E, 1)
